# baseline (device time: 172943 ns/iter reference)
import functools

import jax
import jax.numpy as jnp
from jax import lax
from jax.experimental import pallas as pl
from jax.experimental.pallas import tpu as pltpu

N_DEV = 8
N_HOP = N_DEV - 1
SUBS = 4


def kernel(x):
    _, m, n_total = x.shape
    n_per = n_total // N_DEV
    m_half = m // 2
    sub_m = m_half // SUBS

    def body(
        x_ref,
        out_ref,
        send_f,
        send_b,
        recv_f,
        recv_b,
        stage_f,
        stage_b,
        sf_sems,
        rf_sems,
        sb_sems,
        rb_sems,
        stf_sems,
        stb_sems,
        fill_sems,
        credit_f,
        credit_b,
    ):
        my = lax.axis_index("i")
        left = lax.rem(my + N_DEV - 1, N_DEV)
        right = lax.rem(my + 1, N_DEV)

        def cols(c):
            return pl.ds(c * n_per, n_per)

        def chunk_f(h):
            return lax.rem(my + 2 * N_DEV - h - 2, N_DEV)

        def chunk_b(h):
            return lax.rem(my + h + 2, N_DEV)

        def stage_hop(h):
            slot = h % 2
            f = pltpu.make_async_copy(
                x_ref.at[0, 0:m_half, cols(chunk_f(h))],
                stage_f.at[slot],
                stf_sems.at[slot],
            )
            b = pltpu.make_async_copy(
                x_ref.at[0, m_half:m, cols(chunk_b(h))],
                stage_b.at[slot],
                stb_sems.at[slot],
            )
            f.start()
            b.start()
            return f, b

        def mk(h, s, fwd):
            slot = (h + 1) % 2
            src = send_f if fwd else send_b
            dst = recv_f if fwd else recv_b
            ssem = sf_sems if fwd else sb_sems
            rsem = rf_sems if fwd else rb_sems
            return pltpu.make_async_remote_copy(
                src_ref=src.at[pl.ds(s * sub_m, sub_m), :],
                dst_ref=dst.at[slot, pl.ds(s * sub_m, sub_m), :],
                send_sem=ssem.at[slot, s],
                recv_sem=rsem.at[slot, s],
                device_id=(right if fwd else left,),
                device_id_type=pl.DeviceIdType.MESH,
            )

        fill_f = pltpu.make_async_copy(
            x_ref.at[0, 0:m_half, cols(left)], send_f, fill_sems.at[0]
        )
        fill_b = pltpu.make_async_copy(
            x_ref.at[0, m_half:m, cols(right)], send_b, fill_sems.at[1]
        )
        fill_f.start()
        fill_b.start()
        stage = stage_hop(0)

        barrier_sem = pltpu.get_barrier_semaphore()
        for nbr in (left, right):
            pl.semaphore_signal(
                barrier_sem, inc=1,
                device_id=(nbr,), device_id_type=pl.DeviceIdType.MESH,
            )
        pl.semaphore_wait(barrier_sem, 2)
        fill_f.wait()
        fill_b.wait()

        inflight = {}
        for s in range(SUBS):
            inflight[(0, s, True)] = mk(0, s, True)
            inflight[(0, s, False)] = mk(0, s, False)
            inflight[(0, s, True)].start()
            inflight[(0, s, False)].start()

        for h in range(N_HOP):
            slot = (h + 1) % 2
            st_slot = h % 2
            if h + 1 < N_HOP:
                next_stage = stage_hop(h + 1)
            stage[0].wait()
            stage[1].wait()

            for s in range(SUBS):
                rows = pl.ds(s * sub_m, sub_m)
                out_rows_b = pl.ds(m_half + s * sub_m, sub_m)
                for fwd in (True, False):
                    rdma = inflight.pop((h, s, fwd))
                    rdma.wait_send()
                    rdma.wait_recv()
                    recv = recv_f if fwd else recv_b
                    st = stage_f if fwd else stage_b
                    if h < N_HOP - 1:
                        dst = send_f if fwd else send_b
                        dst[rows, :] = recv[slot, rows, :] + st[st_slot, rows, :]
                    else:
                        orow = rows if fwd else out_rows_b
                        out_ref[orow, :] = recv[slot, rows, :] + st[st_slot, rows, :]
                    if h <= N_HOP - 3:
                        pl.semaphore_signal(
                            (credit_f if fwd else credit_b).at[s],
                            inc=1,
                            device_id=(left if fwd else right,),
                            device_id_type=pl.DeviceIdType.MESH,
                        )
                    if h + 1 < N_HOP:
                        if h + 1 >= 2:
                            pl.semaphore_wait(
                                (credit_f if fwd else credit_b).at[s], 1
                            )
                        nxt = mk(h + 1, s, fwd)
                        inflight[(h + 1, s, fwd)] = nxt
                        nxt.start()

            if h + 1 < N_HOP:
                stage = next_stage

        @functools.partial(
            pl.run_scoped, second_barrier=pltpu.SemaphoreType.REGULAR
        )
        def _(second_barrier):
            for nbr in (left, right):
                pl.semaphore_signal(
                    second_barrier, inc=1,
                    device_id=(nbr,), device_id_type=pl.DeviceIdType.MESH,
                )
            pl.semaphore_wait(second_barrier, 2)

    return pl.pallas_call(
        body,
        out_shape=jax.ShapeDtypeStruct((m, n_per), x.dtype),
        in_specs=[pl.BlockSpec(memory_space=pl.ANY)],
        out_specs=pl.BlockSpec(memory_space=pltpu.VMEM),
        scratch_shapes=[
            pltpu.VMEM((m_half, n_per), x.dtype),
            pltpu.VMEM((m_half, n_per), x.dtype),
            pltpu.VMEM((2, m_half, n_per), x.dtype),
            pltpu.VMEM((2, m_half, n_per), x.dtype),
            pltpu.VMEM((2, m_half, n_per), x.dtype),
            pltpu.VMEM((2, m_half, n_per), x.dtype),
            pltpu.SemaphoreType.DMA((2, SUBS)),
            pltpu.SemaphoreType.DMA((2, SUBS)),
            pltpu.SemaphoreType.DMA((2, SUBS)),
            pltpu.SemaphoreType.DMA((2, SUBS)),
            pltpu.SemaphoreType.DMA((2,)),
            pltpu.SemaphoreType.DMA((2,)),
            pltpu.SemaphoreType.DMA((2,)),
            pltpu.SemaphoreType.REGULAR((SUBS,)),
            pltpu.SemaphoreType.REGULAR((SUBS,)),
        ],
        compiler_params=pltpu.CompilerParams(
            collective_id=0,
            vmem_limit_bytes=60 * 1024 * 1024,
        ),
    )(x)


# device time: 172806 ns/iter; 1.0008x vs baseline; 1.0008x over previous
import functools

import jax
import jax.numpy as jnp
from jax import lax
from jax.experimental import pallas as pl
from jax.experimental.pallas import tpu as pltpu

N_DEV = 8
N_HOP = N_DEV - 1
SUBS = 2


def kernel(x):
    _, m, n_total = x.shape
    n_per = n_total // N_DEV
    m_half = m // 2
    sub_m = m_half // SUBS

    def body(
        x_ref,
        out_ref,
        send_f,
        send_b,
        recv_f,
        recv_b,
        stage_f,
        stage_b,
        sf_sems,
        rf_sems,
        sb_sems,
        rb_sems,
        stf_sems,
        stb_sems,
        fill_sems,
        credit_f,
        credit_b,
    ):
        my = lax.axis_index("i")
        left = lax.rem(my + N_DEV - 1, N_DEV)
        right = lax.rem(my + 1, N_DEV)

        def cols(c):
            return pl.ds(c * n_per, n_per)

        def chunk_f(h):
            return lax.rem(my + 2 * N_DEV - h - 2, N_DEV)

        def chunk_b(h):
            return lax.rem(my + h + 2, N_DEV)

        def stage_hop(h):
            slot = h % 2
            f = pltpu.make_async_copy(
                x_ref.at[0, 0:m_half, cols(chunk_f(h))],
                stage_f.at[slot],
                stf_sems.at[slot],
            )
            b = pltpu.make_async_copy(
                x_ref.at[0, m_half:m, cols(chunk_b(h))],
                stage_b.at[slot],
                stb_sems.at[slot],
            )
            f.start()
            b.start()
            return f, b

        def mk(h, s, fwd):
            slot = (h + 1) % 2
            src = send_f if fwd else send_b
            dst = recv_f if fwd else recv_b
            ssem = sf_sems if fwd else sb_sems
            rsem = rf_sems if fwd else rb_sems
            return pltpu.make_async_remote_copy(
                src_ref=src.at[pl.ds(s * sub_m, sub_m), :],
                dst_ref=dst.at[slot, pl.ds(s * sub_m, sub_m), :],
                send_sem=ssem.at[slot, s],
                recv_sem=rsem.at[slot, s],
                device_id=(right if fwd else left,),
                device_id_type=pl.DeviceIdType.MESH,
            )

        fill_f = pltpu.make_async_copy(
            x_ref.at[0, 0:m_half, cols(left)], send_f, fill_sems.at[0]
        )
        fill_b = pltpu.make_async_copy(
            x_ref.at[0, m_half:m, cols(right)], send_b, fill_sems.at[1]
        )
        fill_f.start()
        fill_b.start()
        stage = stage_hop(0)

        barrier_sem = pltpu.get_barrier_semaphore()
        for nbr in (left, right):
            pl.semaphore_signal(
                barrier_sem, inc=1,
                device_id=(nbr,), device_id_type=pl.DeviceIdType.MESH,
            )
        pl.semaphore_wait(barrier_sem, 2)
        fill_f.wait()
        fill_b.wait()

        inflight = {}
        for s in range(SUBS):
            inflight[(0, s, True)] = mk(0, s, True)
            inflight[(0, s, False)] = mk(0, s, False)
            inflight[(0, s, True)].start()
            inflight[(0, s, False)].start()

        for h in range(N_HOP):
            slot = (h + 1) % 2
            st_slot = h % 2
            if h + 1 < N_HOP:
                next_stage = stage_hop(h + 1)
            stage[0].wait()
            stage[1].wait()

            for s in range(SUBS):
                rows = pl.ds(s * sub_m, sub_m)
                out_rows_b = pl.ds(m_half + s * sub_m, sub_m)
                for fwd in (True, False):
                    rdma = inflight.pop((h, s, fwd))
                    rdma.wait_send()
                    rdma.wait_recv()
                    recv = recv_f if fwd else recv_b
                    st = stage_f if fwd else stage_b
                    if h < N_HOP - 1:
                        dst = send_f if fwd else send_b
                        dst[rows, :] = recv[slot, rows, :] + st[st_slot, rows, :]
                    else:
                        orow = rows if fwd else out_rows_b
                        out_ref[orow, :] = recv[slot, rows, :] + st[st_slot, rows, :]
                    if h <= N_HOP - 3:
                        pl.semaphore_signal(
                            (credit_f if fwd else credit_b).at[s],
                            inc=1,
                            device_id=(left if fwd else right,),
                            device_id_type=pl.DeviceIdType.MESH,
                        )
                    if h + 1 < N_HOP:
                        if h + 1 >= 2:
                            pl.semaphore_wait(
                                (credit_f if fwd else credit_b).at[s], 1
                            )
                        nxt = mk(h + 1, s, fwd)
                        inflight[(h + 1, s, fwd)] = nxt
                        nxt.start()

            if h + 1 < N_HOP:
                stage = next_stage

        @functools.partial(
            pl.run_scoped, second_barrier=pltpu.SemaphoreType.REGULAR
        )
        def _(second_barrier):
            for nbr in (left, right):
                pl.semaphore_signal(
                    second_barrier, inc=1,
                    device_id=(nbr,), device_id_type=pl.DeviceIdType.MESH,
                )
            pl.semaphore_wait(second_barrier, 2)

    return pl.pallas_call(
        body,
        out_shape=jax.ShapeDtypeStruct((m, n_per), x.dtype),
        in_specs=[pl.BlockSpec(memory_space=pl.ANY)],
        out_specs=pl.BlockSpec(memory_space=pltpu.VMEM),
        scratch_shapes=[
            pltpu.VMEM((m_half, n_per), x.dtype),
            pltpu.VMEM((m_half, n_per), x.dtype),
            pltpu.VMEM((2, m_half, n_per), x.dtype),
            pltpu.VMEM((2, m_half, n_per), x.dtype),
            pltpu.VMEM((2, m_half, n_per), x.dtype),
            pltpu.VMEM((2, m_half, n_per), x.dtype),
            pltpu.SemaphoreType.DMA((2, SUBS)),
            pltpu.SemaphoreType.DMA((2, SUBS)),
            pltpu.SemaphoreType.DMA((2, SUBS)),
            pltpu.SemaphoreType.DMA((2, SUBS)),
            pltpu.SemaphoreType.DMA((2,)),
            pltpu.SemaphoreType.DMA((2,)),
            pltpu.SemaphoreType.DMA((2,)),
            pltpu.SemaphoreType.REGULAR((SUBS,)),
            pltpu.SemaphoreType.REGULAR((SUBS,)),
        ],
        compiler_params=pltpu.CompilerParams(
            collective_id=0,
            vmem_limit_bytes=60 * 1024 * 1024,
        ),
    )(x)


# device time: 124969 ns/iter; 1.3839x vs baseline; 1.3828x over previous
import functools

import jax
import jax.numpy as jnp
from jax import lax
from jax.experimental import pallas as pl
from jax.experimental.pallas import tpu as pltpu

N_DEV = 8
ROW_SPLITS = (0, 688, 1368, 2048)


def kernel(x):
    _, m, n_total = x.shape
    n_per = n_total // N_DEV

    def body(x_ref, out_ref, *scratch):
        bufs = [scratch[4 * o : 4 * o + 4] for o in range(3)]
        sems = [scratch[12 + 7 * o : 12 + 7 * o + 7] for o in range(3)]

        my = lax.axis_index("i")
        mz = my // 4
        rr = lax.rem(my, 4)
        myy = rr // 2
        mxx = lax.rem(lax.rem(rr, 2) + myy, 2)

        def pos(xx, yy, zz):
            return 4 * zz + 2 * yy + lax.rem(xx + yy, 2)

        def cols(c):
            return pl.ds(c * n_per, n_per)

        ords = []
        for o, (d1, d2, d3) in enumerate([("x", "y", "z"),
                                          ("y", "z", "x"),
                                          ("z", "x", "y")]):
            coord = {"x": mxx, "y": myy, "z": mz}

            def cpos(d_vals):
                return pos(d_vals["x"], d_vals["y"], d_vals["z"])

            def with_(base, **kw):
                d = dict(base)
                d.update(kw)
                return d

            mine = coord
            p1 = cpos(with_(mine, **{d1: 1 - coord[d1]}))
            p2 = cpos(with_(mine, **{d2: 1 - coord[d2]}))
            p3 = cpos(with_(mine, **{d3: 1 - coord[d3]}))

            def dest_chunk(k, _d1=d1, _d2=d2, _d3=d3, _c=coord):
                return cpos(with_(_c, **{_d1: 1 - _c[_d1],
                                         _d2: k // 2, _d3: k % 2}))

            def loc_chunk(k, _d1=d1, _d2=d2, _d3=d3, _c=coord):
                return cpos(with_(_c, **{_d2: k // 2, _d3: k % 2}))

            rs, re = ROW_SPLITS[o], ROW_SPLITS[o + 1]
            ords.append(dict(
                rs=rs, nr=re - rs,
                p1=p1, p2=p2, p3=p3,
                dest=dest_chunk, loc=loc_chunk,
                keep2=coord[d2], keep3=coord[d3],
            ))

        stage_copies = []
        for o, od in enumerate(ords):
            A1, R1, R2, R3 = bufs[o]
            st, s1, r1, s2, r2, s3, r3 = sems[o]
            cps = []
            for k in range(4):
                cp = pltpu.make_async_copy(
                    x_ref.at[0, pl.ds(od["rs"], od["nr"]), cols(od["loc"](k))],
                    A1.at[:, pl.ds(k * n_per, n_per)],
                    st.at[k],
                )
                cp.start()
                cps.append(cp)
            stage_copies.append(cps)

        barrier_sem = pltpu.get_barrier_semaphore()
        for od in ords:
            pl.semaphore_signal(
                barrier_sem, inc=1,
                device_id=(od["p1"],), device_id_type=pl.DeviceIdType.MESH,
            )
        pl.semaphore_wait(barrier_sem, 3)

        rd1 = []
        for o, od in enumerate(ords):
            A1, R1, R2, R3 = bufs[o]
            st, s1, r1, s2, r2, s3, r3 = sems[o]
            rds = []
            for k in range(4):
                rd = pltpu.make_async_remote_copy(
                    src_ref=x_ref.at[0, pl.ds(od["rs"], od["nr"]),
                                     cols(od["dest"](k))],
                    dst_ref=R1.at[:, pl.ds(k * n_per, n_per)],
                    send_sem=s1.at[k],
                    recv_sem=r1.at[k],
                    device_id=(od["p1"],),
                    device_id_type=pl.DeviceIdType.MESH,
                )
                rd.start()
                rds.append(rd)
            rd1.append(rds)

        rd2 = []
        for o, od in enumerate(ords):
            A1, R1, R2, R3 = bufs[o]
            st, s1, r1, s2, r2, s3, r3 = sems[o]
            for k in range(4):
                stage_copies[o][k].wait()
                rd1[o][k].wait_send()
                rd1[o][k].wait_recv()
            A1[...] = A1[...] + R1[...]
            send2 = pl.ds((1 - od["keep2"]) * 2 * n_per, 2 * n_per)
            rd = pltpu.make_async_remote_copy(
                src_ref=A1.at[:, send2],
                dst_ref=R2,
                send_sem=s2,
                recv_sem=r2,
                device_id=(od["p2"],),
                device_id_type=pl.DeviceIdType.MESH,
            )
            rd.start()
            rd2.append(rd)

        rd3 = []
        for o, od in enumerate(ords):
            A1, R1, R2, R3 = bufs[o]
            st, s1, r1, s2, r2, s3, r3 = sems[o]
            rd2[o].wait_send()
            rd2[o].wait_recv()
            keep2 = pl.ds(od["keep2"] * 2 * n_per, 2 * n_per)
            A1[:, keep2] = A1[:, keep2] + R2[...]
            send3 = pl.ds(od["keep2"] * 2 * n_per + (1 - od["keep3"]) * n_per,
                          n_per)
            rd = pltpu.make_async_remote_copy(
                src_ref=A1.at[:, send3],
                dst_ref=R3,
                send_sem=s3,
                recv_sem=r3,
                device_id=(od["p3"],),
                device_id_type=pl.DeviceIdType.MESH,
            )
            rd.start()
            rd3.append(rd)

        for o, od in enumerate(ords):
            A1, R1, R2, R3 = bufs[o]
            rd3[o].wait_send()
            rd3[o].wait_recv()
            fin = pl.ds(od["keep2"] * 2 * n_per + od["keep3"] * n_per, n_per)
            out_ref[pl.ds(od["rs"], od["nr"]), :] = A1[:, fin] + R3[...]

        @functools.partial(
            pl.run_scoped, second_barrier=pltpu.SemaphoreType.REGULAR
        )
        def _(second_barrier):
            for od in ords:
                pl.semaphore_signal(
                    second_barrier, inc=1,
                    device_id=(od["p1"],), device_id_type=pl.DeviceIdType.MESH,
                )
            pl.semaphore_wait(second_barrier, 3)

    scratch_shapes = []
    for o in range(3):
        nr = ROW_SPLITS[o + 1] - ROW_SPLITS[o]
        scratch_shapes += [
            pltpu.VMEM((nr, 4 * n_per), x.dtype),
            pltpu.VMEM((nr, 4 * n_per), x.dtype),
            pltpu.VMEM((nr, 2 * n_per), x.dtype),
            pltpu.VMEM((nr, 1 * n_per), x.dtype),
        ]
    for o in range(3):
        scratch_shapes += [
            pltpu.SemaphoreType.DMA((4,)),
            pltpu.SemaphoreType.DMA((4,)),
            pltpu.SemaphoreType.DMA((4,)),
            pltpu.SemaphoreType.DMA,
            pltpu.SemaphoreType.DMA,
            pltpu.SemaphoreType.DMA,
            pltpu.SemaphoreType.DMA,
        ]

    return pl.pallas_call(
        body,
        out_shape=jax.ShapeDtypeStruct((m, n_per), x.dtype),
        in_specs=[pl.BlockSpec(memory_space=pl.ANY)],
        out_specs=pl.BlockSpec(memory_space=pltpu.VMEM),
        scratch_shapes=scratch_shapes,
        compiler_params=pltpu.CompilerParams(
            collective_id=0,
            vmem_limit_bytes=60 * 1024 * 1024,
        ),
    )(x)


# device time: 119709 ns/iter; 1.4447x vs baseline; 1.0439x over previous
import functools

import jax
import jax.numpy as jnp
from jax import lax
from jax.experimental import pallas as pl
from jax.experimental.pallas import tpu as pltpu

N_DEV = 8
ROW_SPLITS = (0, 688, 1368, 2048)


def kernel(x):
    _, m, n_total = x.shape
    n_per = n_total // N_DEV

    def body(x_ref, out_ref, *scratch):
        bufs = [scratch[4 * o : 4 * o + 4] for o in range(3)]
        sems = [scratch[12 + 7 * o : 12 + 7 * o + 7] for o in range(3)]

        my = lax.axis_index("i")
        mz = my // 4
        rr = lax.rem(my, 4)
        myy = rr // 2
        mxx = lax.rem(lax.rem(rr, 2) + myy, 2)

        def pos(xx, yy, zz):
            return 4 * zz + 2 * yy + lax.rem(xx + yy, 2)

        def cols(c):
            return pl.ds(c * n_per, n_per)

        ords = []
        for o, (d1, d2, d3) in enumerate([("x", "y", "z"),
                                          ("y", "z", "x"),
                                          ("z", "x", "y")]):
            coord = {"x": mxx, "y": myy, "z": mz}

            def cpos(d_vals):
                return pos(d_vals["x"], d_vals["y"], d_vals["z"])

            def with_(base, **kw):
                d = dict(base)
                d.update(kw)
                return d

            p1 = cpos(with_(coord, **{d1: 1 - coord[d1]}))
            p2 = cpos(with_(coord, **{d2: 1 - coord[d2]}))
            p3 = cpos(with_(coord, **{d3: 1 - coord[d3]}))

            def dest_chunk(k, _d1=d1, _d2=d2, _d3=d3, _c=coord):
                return cpos(with_(_c, **{_d1: 1 - _c[_d1],
                                         _d2: k // 2, _d3: lax.rem(k, 2)}))

            def loc_chunk(k, _d1=d1, _d2=d2, _d3=d3, _c=coord):
                return cpos(with_(_c, **{_d2: k // 2, _d3: lax.rem(k, 2)}))

            c2, c3 = coord[d2], coord[d3]
            kseq = [2 * (1 - c2), 2 * (1 - c2) + 1, 2 * c2, 2 * c2 + 1]
            jseq = [1 - c3, c3]
            rs, re = ROW_SPLITS[o], ROW_SPLITS[o + 1]
            ords.append(dict(
                rs=rs, nr=re - rs,
                p1=p1, p2=p2, p3=p3,
                dest=dest_chunk, loc=loc_chunk,
                c2=c2, c3=c3, kseq=kseq, jseq=jseq,
            ))

        stage_copies = []
        for o, od in enumerate(ords):
            A1 = bufs[o][0]
            st = sems[o][0]
            cps = []
            for k in range(4):
                cp = pltpu.make_async_copy(
                    x_ref.at[0, pl.ds(od["rs"], od["nr"]), cols(od["loc"](k))],
                    A1.at[:, pl.ds(k * n_per, n_per)],
                    st.at[k],
                )
                cp.start()
                cps.append(cp)
            stage_copies.append(cps)

        barrier_sem = pltpu.get_barrier_semaphore()
        for od in ords:
            pl.semaphore_signal(
                barrier_sem, inc=1,
                device_id=(od["p1"],), device_id_type=pl.DeviceIdType.MESH,
            )
        pl.semaphore_wait(barrier_sem, 3)

        rd1 = []
        for o, od in enumerate(ords):
            R1 = bufs[o][1]
            _, s1, r1 = sems[o][0], sems[o][1], sems[o][2]
            rds = []
            for j in range(4):
                rd = pltpu.make_async_remote_copy(
                    src_ref=x_ref.at[0, pl.ds(od["rs"], od["nr"]),
                                     cols(od["dest"](od["kseq"][j]))],
                    dst_ref=R1.at[:, pl.ds(j * n_per, n_per)],
                    send_sem=s1.at[j],
                    recv_sem=r1.at[j],
                    device_id=(od["p1"],),
                    device_id_type=pl.DeviceIdType.MESH,
                )
                rd.start()
                rds.append(rd)
            rd1.append(rds)

        for cps in stage_copies:
            for cp in cps:
                cp.wait()

        rd2 = [None] * 3
        for j in range(4):
            for o, od in enumerate(ords):
                A1, R1, R2, _ = bufs[o]
                rd1[o][j].wait_send()
                rd1[o][j].wait_recv()
                kk = od["kseq"][j]
                sl = pl.ds(kk * n_per, n_per)
                A1[:, sl] = A1[:, sl] + R1[:, pl.ds(j * n_per, n_per)]
                if j == 1:
                    s2, r2 = sems[o][3], sems[o][4]
                    subs = []
                    for j2 in range(2):
                        src_c = (1 - od["c2"]) * 2 * n_per + od["jseq"][j2] * n_per
                        rd = pltpu.make_async_remote_copy(
                            src_ref=A1.at[:, pl.ds(src_c, n_per)],
                            dst_ref=R2.at[:, pl.ds(j2 * n_per, n_per)],
                            send_sem=s2.at[j2],
                            recv_sem=r2.at[j2],
                            device_id=(od["p2"],),
                            device_id_type=pl.DeviceIdType.MESH,
                        )
                        rd.start()
                        subs.append(rd)
                    rd2[o] = subs

        rd3 = [None] * 3
        for j2 in range(2):
            for o, od in enumerate(ords):
                A1, _, R2, R3 = bufs[o]
                rd2[o][j2].wait_send()
                rd2[o][j2].wait_recv()
                dst_c = od["c2"] * 2 * n_per + od["jseq"][j2] * n_per
                sl = pl.ds(dst_c, n_per)
                A1[:, sl] = A1[:, sl] + R2[:, pl.ds(j2 * n_per, n_per)]
                if j2 == 0:
                    s3, r3 = sems[o][5], sems[o][6]
                    rd = pltpu.make_async_remote_copy(
                        src_ref=A1.at[:, sl],
                        dst_ref=R3,
                        send_sem=s3,
                        recv_sem=r3,
                        device_id=(od["p3"],),
                        device_id_type=pl.DeviceIdType.MESH,
                    )
                    rd.start()
                    rd3[o] = rd

        for o, od in enumerate(ords):
            A1, _, _, R3 = bufs[o]
            rd3[o].wait_send()
            rd3[o].wait_recv()
            fin = pl.ds(od["c2"] * 2 * n_per + od["c3"] * n_per, n_per)
            out_ref[pl.ds(od["rs"], od["nr"]), :] = A1[:, fin] + R3[...]

        @functools.partial(
            pl.run_scoped, second_barrier=pltpu.SemaphoreType.REGULAR
        )
        def _(second_barrier):
            for od in ords:
                pl.semaphore_signal(
                    second_barrier, inc=1,
                    device_id=(od["p1"],), device_id_type=pl.DeviceIdType.MESH,
                )
            pl.semaphore_wait(second_barrier, 3)

    scratch_shapes = []
    for o in range(3):
        nr = ROW_SPLITS[o + 1] - ROW_SPLITS[o]
        scratch_shapes += [
            pltpu.VMEM((nr, 4 * n_per), x.dtype),
            pltpu.VMEM((nr, 4 * n_per), x.dtype),
            pltpu.VMEM((nr, 2 * n_per), x.dtype),
            pltpu.VMEM((nr, 1 * n_per), x.dtype),
        ]
    for o in range(3):
        scratch_shapes += [
            pltpu.SemaphoreType.DMA((4,)),
            pltpu.SemaphoreType.DMA((4,)),
            pltpu.SemaphoreType.DMA((4,)),
            pltpu.SemaphoreType.DMA((2,)),
            pltpu.SemaphoreType.DMA((2,)),
            pltpu.SemaphoreType.DMA,
            pltpu.SemaphoreType.DMA,
        ]

    return pl.pallas_call(
        body,
        out_shape=jax.ShapeDtypeStruct((m, n_per), x.dtype),
        in_specs=[pl.BlockSpec(memory_space=pl.ANY)],
        out_specs=pl.BlockSpec(memory_space=pltpu.VMEM),
        scratch_shapes=scratch_shapes,
        compiler_params=pltpu.CompilerParams(
            collective_id=0,
            vmem_limit_bytes=60 * 1024 * 1024,
        ),
    )(x)
